# baseline (device time: 127879 ns/iter reference)
import jax
import jax.numpy as jnp
from jax import lax
from jax.experimental import pallas as pl
from jax.experimental.pallas import tpu as pltpu

B, S, D = 2, 512, 2048
DC_HALF = 128
H, DH, DR = 16, 128, 32
SCALE = (DH + DR) ** -0.5


def _dot(a, b, contract=((1,), (0,))):
    return lax.dot_general(
        a, b, (contract, ((), ())), preferred_element_type=jnp.float32
    )


def _kvq_body(x_ref, wdkv_ref, wuk_ref, wuv_ref, wq_ref, wqr_ref, wkr_ref,
              k_ref, v_ref, q_ref, qr_ref, kr_ref,
              c_loc, c_rcv, wuk_rcv, wuv_rcv, send_sems, recv_sems):
    my_x = lax.axis_index("x")
    peer = (1 - my_x, lax.axis_index("y"), lax.axis_index("z"))

    barrier = pltpu.get_barrier_semaphore()
    pl.semaphore_signal(barrier, inc=1, device_id=peer,
                        device_id_type=pl.DeviceIdType.MESH)
    pl.semaphore_wait(barrier, 1)

    for b in range(B):
        c_loc[b, :, :] = _dot(x_ref[b], wdkv_ref[...])

    rdmas = []
    for i, (src, dst) in enumerate(
        [(c_loc, c_rcv), (wuk_ref, wuk_rcv), (wuv_ref, wuv_rcv)]
    ):
        rdma = pltpu.make_async_remote_copy(
            src_ref=src, dst_ref=dst,
            send_sem=send_sems.at[i], recv_sem=recv_sems.at[i],
            device_id=peer, device_id_type=pl.DeviceIdType.MESH,
        )
        rdma.start()
        rdmas.append(rdma)

    for b in range(B):
        xb = x_ref[b]
        q_ref[b, :, :] = _dot(xb, wq_ref[...])
        qr_ref[b, :, :] = _dot(xb, wqr_ref[...])
        kr_ref[b, :, :] = _dot(xb, wkr_ref[...])
        k_ref[b, :, :] = _dot(c_loc[b], wuk_ref[...])
        v_ref[b, :, :] = _dot(c_loc[b], wuv_ref[...])

    for rdma in rdmas:
        rdma.wait()

    for b in range(B):
        k_ref[b, :, :] = k_ref[b] + _dot(c_rcv[b], wuk_rcv[...])
        v_ref[b, :, :] = v_ref[b] + _dot(c_rcv[b], wuv_rcv[...])


def _attn_out_body(q_ref, k_ref, v_ref, qr_ref, kr_ref, wo_ref, y_ref):
    kr = kr_ref[0]
    for h in range(H):
        q = q_ref[0, :, h * DH:(h + 1) * DH]
        k = k_ref[0, :, h * DH:(h + 1) * DH]
        v = v_ref[0, :, h * DH:(h + 1) * DH]
        qr = qr_ref[0, :, h * DR:(h + 1) * DR]
        s = (_dot(q, k, ((1,), (1,))) + _dot(qr, kr, ((1,), (1,)))) * SCALE
        m = jnp.max(s, axis=-1, keepdims=True)
        p = jnp.exp(s - m)
        p = p / jnp.sum(p, axis=-1, keepdims=True)
        o_h = _dot(p, v, ((1,), (0,)))
        contrib = _dot(o_h, wo_ref[h * DH:(h + 1) * DH, :])
        if h == 0:
            y_ref[0, :, :] = contrib
        else:
            y_ref[0, :, :] = y_ref[0] + contrib


def kernel(x, Wdkv, Wuk, Wuv, Wq, Wqr, Wkr, Wo):
    f32 = jnp.float32
    vmem = pl.BlockSpec(memory_space=pltpu.VMEM)

    k, v, q, qr, kr = pl.pallas_call(
        _kvq_body,
        out_shape=[
            jax.ShapeDtypeStruct((B, S, D), f32),
            jax.ShapeDtypeStruct((B, S, D), f32),
            jax.ShapeDtypeStruct((B, S, D), f32),
            jax.ShapeDtypeStruct((B, S, H * DR), f32),
            jax.ShapeDtypeStruct((B, S, DR), f32),
        ],
        in_specs=[vmem] * 7,
        out_specs=[vmem] * 5,
        scratch_shapes=[
            pltpu.VMEM((B, S, DC_HALF), f32),
            pltpu.VMEM((B, S, DC_HALF), f32),
            pltpu.VMEM((DC_HALF, D), f32),
            pltpu.VMEM((DC_HALF, D), f32),
            pltpu.SemaphoreType.DMA((3,)),
            pltpu.SemaphoreType.DMA((3,)),
        ],
        compiler_params=pltpu.CompilerParams(
            collective_id=0, vmem_limit_bytes=100 * 1024 * 1024
        ),
    )(x, Wdkv, Wuk, Wuv, Wq, Wqr, Wkr)

    return pl.pallas_call(
        _attn_out_body,
        grid=(B,),
        in_specs=[
            pl.BlockSpec((1, S, D), lambda b: (b, 0, 0)),
            pl.BlockSpec((1, S, D), lambda b: (b, 0, 0)),
            pl.BlockSpec((1, S, D), lambda b: (b, 0, 0)),
            pl.BlockSpec((1, S, H * DR), lambda b: (b, 0, 0)),
            pl.BlockSpec((1, S, DR), lambda b: (b, 0, 0)),
            pl.BlockSpec((D, D), lambda b: (0, 0)),
        ],
        out_specs=pl.BlockSpec((1, S, D), lambda b: (b, 0, 0)),
        out_shape=jax.ShapeDtypeStruct((B, S, D), f32),
        compiler_params=pltpu.CompilerParams(
            vmem_limit_bytes=100 * 1024 * 1024
        ),
    )(q, k, v, qr, kr, Wo)


# device time: 101320 ns/iter; 1.2621x vs baseline; 1.2621x over previous
import jax
import jax.numpy as jnp
from jax import lax
from jax.experimental import pallas as pl
from jax.experimental.pallas import tpu as pltpu

B, S, D = 2, 512, 2048
DC_HALF = 128
H, DH, DR = 16, 128, 32
SCALE = (DH + DR) ** -0.5


def _dot(a, b, contract=((1,), (0,))):
    return lax.dot_general(
        a, b, (contract, ((), ())), preferred_element_type=jnp.float32
    )


def _kvq_body(x_ref, wdkv_ref, wuk_ref, wuv_ref, wq_ref, wqr_ref, wkr_ref,
              k_ref, v_ref, q_ref, qr_ref, kr_ref,
              c_loc, c_rcv, wuk_rcv, wuv_rcv, send_sems, recv_sems):
    b = pl.program_id(0)
    my_x = lax.axis_index("x")
    peer = (1 - my_x, lax.axis_index("y"), lax.axis_index("z"))

    def step(slot):
        c_loc[slot, :, :] = _dot(x_ref[0], wdkv_ref[...])

        rdmas = [pltpu.make_async_remote_copy(
            src_ref=c_loc.at[slot], dst_ref=c_rcv.at[slot],
            send_sem=send_sems.at[slot], recv_sem=recv_sems.at[slot],
            device_id=peer, device_id_type=pl.DeviceIdType.MESH,
        )]
        if slot == 0:
            for i, (src, dst) in enumerate(
                [(wuk_ref, wuk_rcv), (wuv_ref, wuv_rcv)], start=2
            ):
                rdmas.append(pltpu.make_async_remote_copy(
                    src_ref=src, dst_ref=dst,
                    send_sem=send_sems.at[i], recv_sem=recv_sems.at[i],
                    device_id=peer, device_id_type=pl.DeviceIdType.MESH,
                ))
        for r in rdmas:
            r.start()

        xb = x_ref[0]
        q_ref[0, :, :] = _dot(xb, wq_ref[...])
        qr_ref[0, :, :] = _dot(xb, wqr_ref[...])
        kr_ref[0, :, :] = _dot(xb, wkr_ref[...])
        k_ref[0, :, :] = _dot(c_loc[slot], wuk_ref[...])
        v_ref[0, :, :] = _dot(c_loc[slot], wuv_ref[...])

        for r in rdmas:
            r.wait()

        k_ref[0, :, :] = k_ref[0] + _dot(c_rcv[slot], wuk_rcv[...])
        v_ref[0, :, :] = v_ref[0] + _dot(c_rcv[slot], wuv_rcv[...])

    @pl.when(b == 0)
    def _():
        barrier = pltpu.get_barrier_semaphore()
        pl.semaphore_signal(barrier, inc=1, device_id=peer,
                            device_id_type=pl.DeviceIdType.MESH)
        pl.semaphore_wait(barrier, 1)
        step(0)

    @pl.when(b == 1)
    def _():
        step(1)


def _attn_body(q_ref, k_ref, v_ref, qr_ref, kr_ref, o_ref):
    kr = kr_ref[0]
    for h in range(H):
        q = q_ref[0, :, h * DH:(h + 1) * DH]
        k = k_ref[0, :, h * DH:(h + 1) * DH]
        v = v_ref[0, :, h * DH:(h + 1) * DH]
        qr = qr_ref[0, :, h * DR:(h + 1) * DR]
        s = (_dot(q, k, ((1,), (1,))) + _dot(qr, kr, ((1,), (1,)))) * SCALE
        p = jnp.exp(s)
        denom = jnp.sum(p, axis=-1, keepdims=True)
        o_ref[0, :, h * DH:(h + 1) * DH] = _dot(p, v, ((1,), (0,))) / denom


def _out_body(o_ref, wo_ref, y_ref):
    for b in range(B):
        y_ref[b, :, :] = _dot(o_ref[b], wo_ref[...])


def kernel(x, Wdkv, Wuk, Wuv, Wq, Wqr, Wkr, Wo):
    f32 = jnp.float32
    vmem = pl.BlockSpec(memory_space=pltpu.VMEM)

    k, v, q, qr, kr = pl.pallas_call(
        _kvq_body,
        grid=(B,),
        out_shape=[
            jax.ShapeDtypeStruct((B, S, D), f32),
            jax.ShapeDtypeStruct((B, S, D), f32),
            jax.ShapeDtypeStruct((B, S, D), f32),
            jax.ShapeDtypeStruct((B, S, H * DR), f32),
            jax.ShapeDtypeStruct((B, S, DR), f32),
        ],
        in_specs=[
            pl.BlockSpec((1, S, D), lambda b: (b, 0, 0)),
            pl.BlockSpec((D, DC_HALF), lambda b: (0, 0)),
            pl.BlockSpec((DC_HALF, D), lambda b: (0, 0)),
            pl.BlockSpec((DC_HALF, D), lambda b: (0, 0)),
            pl.BlockSpec((D, D), lambda b: (0, 0)),
            pl.BlockSpec((D, H * DR), lambda b: (0, 0)),
            pl.BlockSpec((D, DR), lambda b: (0, 0)),
        ],
        out_specs=[
            pl.BlockSpec((1, S, D), lambda b: (b, 0, 0)),
            pl.BlockSpec((1, S, D), lambda b: (b, 0, 0)),
            pl.BlockSpec((1, S, D), lambda b: (b, 0, 0)),
            pl.BlockSpec((1, S, H * DR), lambda b: (b, 0, 0)),
            pl.BlockSpec((1, S, DR), lambda b: (b, 0, 0)),
        ],
        scratch_shapes=[
            pltpu.VMEM((B, S, DC_HALF), f32),
            pltpu.VMEM((B, S, DC_HALF), f32),
            pltpu.VMEM((DC_HALF, D), f32),
            pltpu.VMEM((DC_HALF, D), f32),
            pltpu.SemaphoreType.DMA((4,)),
            pltpu.SemaphoreType.DMA((4,)),
        ],
        compiler_params=pltpu.CompilerParams(
            collective_id=0, vmem_limit_bytes=100 * 1024 * 1024,
            dimension_semantics=("arbitrary",),
        ),
    )(x, Wdkv, Wuk, Wuv, Wq, Wqr, Wkr)

    o = pl.pallas_call(
        _attn_body,
        grid=(B,),
        in_specs=[
            pl.BlockSpec((1, S, D), lambda b: (b, 0, 0)),
            pl.BlockSpec((1, S, D), lambda b: (b, 0, 0)),
            pl.BlockSpec((1, S, D), lambda b: (b, 0, 0)),
            pl.BlockSpec((1, S, H * DR), lambda b: (b, 0, 0)),
            pl.BlockSpec((1, S, DR), lambda b: (b, 0, 0)),
        ],
        out_specs=pl.BlockSpec((1, S, D), lambda b: (b, 0, 0)),
        out_shape=jax.ShapeDtypeStruct((B, S, D), f32),
    )(q, k, v, qr, kr)

    return pl.pallas_call(
        _out_body,
        out_shape=jax.ShapeDtypeStruct((B, S, D), f32),
        in_specs=[vmem] * 2,
        out_specs=vmem,
    )(o, Wo)


# device time: 87489 ns/iter; 1.4617x vs baseline; 1.1581x over previous
import jax
import jax.numpy as jnp
from jax import lax
from jax.experimental import pallas as pl
from jax.experimental.pallas import tpu as pltpu

B, S, D = 2, 512, 2048
DC_HALF = 128
H, DH, DR = 16, 128, 32
SCALE = (DH + DR) ** -0.5


def _dot(a, b, contract=((1,), (0,))):
    return lax.dot_general(
        a, b, (contract, ((), ())), preferred_element_type=jnp.float32
    )


def _kvq_body(x_ref, wdkv_ref, wuk_ref, wuv_ref, wq_ref, wqr_ref, wkr_ref,
              k_ref, v_ref, q_ref, qr_ref, kr_ref,
              c_loc, c_snd, c_rcv, wuk_snd, wuk_rcv, wuv_snd, wuv_rcv,
              send_sems, recv_sems):
    b = pl.program_id(0)
    my_x = lax.axis_index("x")
    peer = (1 - my_x, lax.axis_index("y"), lax.axis_index("z"))

    def step(slot):
        rdmas = []
        if slot == 0:
            wuk_snd[...] = wuk_ref[...].astype(jnp.bfloat16)
            wuv_snd[...] = wuv_ref[...].astype(jnp.bfloat16)
            for i, (src, dst) in enumerate(
                [(wuk_snd, wuk_rcv), (wuv_snd, wuv_rcv)], start=2
            ):
                rdmas.append(pltpu.make_async_remote_copy(
                    src_ref=src, dst_ref=dst,
                    send_sem=send_sems.at[i], recv_sem=recv_sems.at[i],
                    device_id=peer, device_id_type=pl.DeviceIdType.MESH,
                ))
            for r in rdmas:
                r.start()

        c_loc[slot, :, :] = _dot(x_ref[0], wdkv_ref[...])
        c_snd[slot, :, :] = c_loc[slot].astype(jnp.bfloat16)
        c_rdma = pltpu.make_async_remote_copy(
            src_ref=c_snd.at[slot], dst_ref=c_rcv.at[slot],
            send_sem=send_sems.at[slot], recv_sem=recv_sems.at[slot],
            device_id=peer, device_id_type=pl.DeviceIdType.MESH,
        )
        c_rdma.start()
        rdmas.append(c_rdma)

        xb = x_ref[0]
        q_ref[0, :, :] = _dot(xb, wq_ref[...])
        qr_ref[0, :, :] = _dot(xb, wqr_ref[...])
        kr_ref[0, :, :] = _dot(xb, wkr_ref[...])
        k_ref[0, :, :] = _dot(c_loc[slot], wuk_ref[...])
        v_ref[0, :, :] = _dot(c_loc[slot], wuv_ref[...])

        for r in rdmas:
            r.wait()

        c_r = c_rcv[slot].astype(jnp.float32)
        k_ref[0, :, :] = k_ref[0] + _dot(c_r, wuk_rcv[...].astype(jnp.float32))
        v_ref[0, :, :] = v_ref[0] + _dot(c_r, wuv_rcv[...].astype(jnp.float32))

    @pl.when(b == 0)
    def _():
        barrier = pltpu.get_barrier_semaphore()
        pl.semaphore_signal(barrier, inc=1, device_id=peer,
                            device_id_type=pl.DeviceIdType.MESH)
        pl.semaphore_wait(barrier, 1)
        step(0)

    @pl.when(b == 1)
    def _():
        step(1)


def _attn_body(q_ref, k_ref, v_ref, qr_ref, kr_ref, o_ref):
    kr = kr_ref[0]
    for h in range(H):
        q = q_ref[0, :, h * DH:(h + 1) * DH]
        k = k_ref[0, :, h * DH:(h + 1) * DH]
        v = v_ref[0, :, h * DH:(h + 1) * DH]
        qr = qr_ref[0, :, h * DR:(h + 1) * DR]
        s = (_dot(q, k, ((1,), (1,))) + _dot(qr, kr, ((1,), (1,)))) * SCALE
        p = jnp.exp(s)
        denom = jnp.sum(p, axis=-1, keepdims=True)
        o_ref[0, :, h * DH:(h + 1) * DH] = _dot(p, v, ((1,), (0,))) / denom


def _out_body(o_ref, wo_ref, y_ref):
    for b in range(B):
        y_ref[b, :, :] = _dot(o_ref[b], wo_ref[...])


def kernel(x, Wdkv, Wuk, Wuv, Wq, Wqr, Wkr, Wo):
    f32 = jnp.float32
    vmem = pl.BlockSpec(memory_space=pltpu.VMEM)

    k, v, q, qr, kr = pl.pallas_call(
        _kvq_body,
        grid=(B,),
        out_shape=[
            jax.ShapeDtypeStruct((B, S, D), f32),
            jax.ShapeDtypeStruct((B, S, D), f32),
            jax.ShapeDtypeStruct((B, S, D), f32),
            jax.ShapeDtypeStruct((B, S, H * DR), f32),
            jax.ShapeDtypeStruct((B, S, DR), f32),
        ],
        in_specs=[
            pl.BlockSpec((1, S, D), lambda b: (b, 0, 0)),
            pl.BlockSpec((D, DC_HALF), lambda b: (0, 0)),
            pl.BlockSpec((DC_HALF, D), lambda b: (0, 0)),
            pl.BlockSpec((DC_HALF, D), lambda b: (0, 0)),
            pl.BlockSpec((D, D), lambda b: (0, 0)),
            pl.BlockSpec((D, H * DR), lambda b: (0, 0)),
            pl.BlockSpec((D, DR), lambda b: (0, 0)),
        ],
        out_specs=[
            pl.BlockSpec((1, S, D), lambda b: (b, 0, 0)),
            pl.BlockSpec((1, S, D), lambda b: (b, 0, 0)),
            pl.BlockSpec((1, S, D), lambda b: (b, 0, 0)),
            pl.BlockSpec((1, S, H * DR), lambda b: (b, 0, 0)),
            pl.BlockSpec((1, S, DR), lambda b: (b, 0, 0)),
        ],
        scratch_shapes=[
            pltpu.VMEM((B, S, DC_HALF), f32),
            pltpu.VMEM((B, S, DC_HALF), jnp.bfloat16),
            pltpu.VMEM((B, S, DC_HALF), jnp.bfloat16),
            pltpu.VMEM((DC_HALF, D), jnp.bfloat16),
            pltpu.VMEM((DC_HALF, D), jnp.bfloat16),
            pltpu.VMEM((DC_HALF, D), jnp.bfloat16),
            pltpu.VMEM((DC_HALF, D), jnp.bfloat16),
            pltpu.SemaphoreType.DMA((4,)),
            pltpu.SemaphoreType.DMA((4,)),
        ],
        compiler_params=pltpu.CompilerParams(
            collective_id=0, vmem_limit_bytes=100 * 1024 * 1024,
            dimension_semantics=("arbitrary",),
        ),
    )(x, Wdkv, Wuk, Wuv, Wq, Wqr, Wkr)

    o = pl.pallas_call(
        _attn_body,
        grid=(B,),
        in_specs=[
            pl.BlockSpec((1, S, D), lambda b: (b, 0, 0)),
            pl.BlockSpec((1, S, D), lambda b: (b, 0, 0)),
            pl.BlockSpec((1, S, D), lambda b: (b, 0, 0)),
            pl.BlockSpec((1, S, H * DR), lambda b: (b, 0, 0)),
            pl.BlockSpec((1, S, DR), lambda b: (b, 0, 0)),
        ],
        out_specs=pl.BlockSpec((1, S, D), lambda b: (b, 0, 0)),
        out_shape=jax.ShapeDtypeStruct((B, S, D), f32),
    )(q, k, v, qr, kr)

    return pl.pallas_call(
        _out_body,
        out_shape=jax.ShapeDtypeStruct((B, S, D), f32),
        in_specs=[vmem] * 2,
        out_specs=vmem,
    )(o, Wo)


# device time: 76943 ns/iter; 1.6620x vs baseline; 1.1371x over previous
import jax
import jax.numpy as jnp
from jax import lax
from jax.experimental import pallas as pl
from jax.experimental.pallas import tpu as pltpu

B, S, D = 2, 512, 2048
DC_HALF = 128
H, DH, DR = 16, 128, 32
SCALE = (DH + DR) ** -0.5


def _dot(a, b, contract=((1,), (0,))):
    return lax.dot_general(
        a, b, (contract, ((), ())), preferred_element_type=jnp.float32
    )


def _mla_body(x_ref, wdkv_ref, wuk_ref, wuv_ref, wq_ref, wqr_ref, wkr_ref,
              o_ref,
              k_b, v_b, q_b, qr_b, kr_b,
              c_loc, c_snd, c_rcv, wuk_snd, wuk_rcv, wuv_snd, wuv_rcv,
              send_sems, recv_sems):
    b = pl.program_id(0)
    my_x = lax.axis_index("x")
    peer = (1 - my_x, lax.axis_index("y"), lax.axis_index("z"))

    def step(slot):
        rdmas = []
        if slot == 0:
            wuk_snd[...] = wuk_ref[...].astype(jnp.bfloat16)
            wuv_snd[...] = wuv_ref[...].astype(jnp.bfloat16)
            for i, (src, dst) in enumerate(
                [(wuk_snd, wuk_rcv), (wuv_snd, wuv_rcv)], start=2
            ):
                rdmas.append(pltpu.make_async_remote_copy(
                    src_ref=src, dst_ref=dst,
                    send_sem=send_sems.at[i], recv_sem=recv_sems.at[i],
                    device_id=peer, device_id_type=pl.DeviceIdType.MESH,
                ))
            for r in rdmas:
                r.start()

        c_loc[slot, :, :] = _dot(x_ref[0], wdkv_ref[...])
        c_snd[slot, :, :] = c_loc[slot].astype(jnp.bfloat16)
        c_rdma = pltpu.make_async_remote_copy(
            src_ref=c_snd.at[slot], dst_ref=c_rcv.at[slot],
            send_sem=send_sems.at[slot], recv_sem=recv_sems.at[slot],
            device_id=peer, device_id_type=pl.DeviceIdType.MESH,
        )
        c_rdma.start()
        rdmas.append(c_rdma)

        xb = x_ref[0]
        q_b[...] = _dot(xb, wq_ref[...])
        qr_b[...] = _dot(xb, wqr_ref[...])
        kr_b[...] = _dot(xb, wkr_ref[...])
        k_b[...] = _dot(c_loc[slot], wuk_ref[...])
        v_b[...] = _dot(c_loc[slot], wuv_ref[...])

        for r in rdmas:
            r.wait()

        c_r = c_rcv[slot].astype(jnp.float32)
        k_b[...] = k_b[...] + _dot(c_r, wuk_rcv[...].astype(jnp.float32))
        v_b[...] = v_b[...] + _dot(c_r, wuv_rcv[...].astype(jnp.float32))

    @pl.when(b == 0)
    def _():
        barrier = pltpu.get_barrier_semaphore()
        pl.semaphore_signal(barrier, inc=1, device_id=peer,
                            device_id_type=pl.DeviceIdType.MESH)
        pl.semaphore_wait(barrier, 1)
        step(0)

    @pl.when(b == 1)
    def _():
        step(1)

    kr = kr_b[...]
    for h in range(H):
        q = q_b[:, h * DH:(h + 1) * DH]
        k = k_b[:, h * DH:(h + 1) * DH]
        v = v_b[:, h * DH:(h + 1) * DH]
        qr = qr_b[:, h * DR:(h + 1) * DR]
        s = (_dot(q, k, ((1,), (1,))) + _dot(qr, kr, ((1,), (1,)))) * SCALE
        p = jnp.exp(s)
        denom = jnp.sum(p, axis=-1, keepdims=True)
        o_ref[0, :, h * DH:(h + 1) * DH] = _dot(p, v, ((1,), (0,))) / denom


def _out_body(o_ref, wo_ref, y_ref):
    for b in range(B):
        y_ref[b, :, :] = _dot(o_ref[b], wo_ref[...])


def kernel(x, Wdkv, Wuk, Wuv, Wq, Wqr, Wkr, Wo):
    f32 = jnp.float32
    bf16 = jnp.bfloat16
    vmem = pl.BlockSpec(memory_space=pltpu.VMEM)

    o = pl.pallas_call(
        _mla_body,
        grid=(B,),
        out_shape=jax.ShapeDtypeStruct((B, S, D), f32),
        in_specs=[
            pl.BlockSpec((1, S, D), lambda b: (b, 0, 0)),
            pl.BlockSpec((D, DC_HALF), lambda b: (0, 0)),
            pl.BlockSpec((DC_HALF, D), lambda b: (0, 0)),
            pl.BlockSpec((DC_HALF, D), lambda b: (0, 0)),
            pl.BlockSpec((D, D), lambda b: (0, 0)),
            pl.BlockSpec((D, H * DR), lambda b: (0, 0)),
            pl.BlockSpec((D, DR), lambda b: (0, 0)),
        ],
        out_specs=pl.BlockSpec((1, S, D), lambda b: (b, 0, 0)),
        scratch_shapes=[
            pltpu.VMEM((S, D), f32),
            pltpu.VMEM((S, D), f32),
            pltpu.VMEM((S, D), f32),
            pltpu.VMEM((S, H * DR), f32),
            pltpu.VMEM((S, DR), f32),
            pltpu.VMEM((B, S, DC_HALF), f32),
            pltpu.VMEM((B, S, DC_HALF), bf16),
            pltpu.VMEM((B, S, DC_HALF), bf16),
            pltpu.VMEM((DC_HALF, D), bf16),
            pltpu.VMEM((DC_HALF, D), bf16),
            pltpu.VMEM((DC_HALF, D), bf16),
            pltpu.VMEM((DC_HALF, D), bf16),
            pltpu.SemaphoreType.DMA((4,)),
            pltpu.SemaphoreType.DMA((4,)),
        ],
        compiler_params=pltpu.CompilerParams(
            collective_id=0, vmem_limit_bytes=100 * 1024 * 1024,
            dimension_semantics=("arbitrary",),
        ),
    )(x, Wdkv, Wuk, Wuv, Wq, Wqr, Wkr)

    return pl.pallas_call(
        _out_body,
        out_shape=jax.ShapeDtypeStruct((B, S, D), f32),
        in_specs=[vmem] * 2,
        out_specs=vmem,
        compiler_params=pltpu.CompilerParams(
            vmem_limit_bytes=100 * 1024 * 1024
        ),
    )(o, Wo)


# device time: 75047 ns/iter; 1.7040x vs baseline; 1.0253x over previous
import jax
import jax.numpy as jnp
from jax import lax
from jax.experimental import pallas as pl
from jax.experimental.pallas import tpu as pltpu

B, S, D = 2, 512, 2048
DC_HALF = 128
H, DH, DR = 16, 128, 32
SCALE = (DH + DR) ** -0.5


def _dot(a, b, contract=((1,), (0,))):
    return lax.dot_general(
        a, b, (contract, ((), ())), preferred_element_type=jnp.float32
    )


def _mla_body(x_ref, wdkv_ref, wuk_ref, wuv_ref, wq_ref, wqr_ref, wkr_ref,
              o_ref,
              k_b, v_b, q_b, qr_b, kr_b,
              c_loc, c_snd, c_rcv, wuk_snd, wuk_rcv, wuv_snd, wuv_rcv,
              send_sems, recv_sems):
    b = pl.program_id(0)
    my_x = lax.axis_index("x")
    peer = (1 - my_x, lax.axis_index("y"), lax.axis_index("z"))

    def step(slot):
        rdmas = []
        if slot == 0:
            wuk_snd[...] = wuk_ref[...].astype(jnp.bfloat16)
            wuv_snd[...] = wuv_ref[...].astype(jnp.bfloat16)
            for i, (src, dst) in enumerate(
                [(wuk_snd, wuk_rcv), (wuv_snd, wuv_rcv)], start=2
            ):
                rdmas.append(pltpu.make_async_remote_copy(
                    src_ref=src, dst_ref=dst,
                    send_sem=send_sems.at[i], recv_sem=recv_sems.at[i],
                    device_id=peer, device_id_type=pl.DeviceIdType.MESH,
                ))
            for r in rdmas:
                r.start()

        c_loc[slot, :, :] = _dot(x_ref[0], wdkv_ref[...])
        c_snd[slot, :, :] = c_loc[slot].astype(jnp.bfloat16)
        c_rdma = pltpu.make_async_remote_copy(
            src_ref=c_snd.at[slot], dst_ref=c_rcv.at[slot],
            send_sem=send_sems.at[slot], recv_sem=recv_sems.at[slot],
            device_id=peer, device_id_type=pl.DeviceIdType.MESH,
        )
        c_rdma.start()
        rdmas.append(c_rdma)

        xb = x_ref[0]
        q_b[...] = _dot(xb, wq_ref[...])
        qr_b[...] = _dot(xb, wqr_ref[...])
        kr_b[...] = _dot(xb, wkr_ref[...])
        k_b[...] = _dot(c_loc[slot], wuk_ref[...])
        v_b[...] = _dot(c_loc[slot], wuv_ref[...])

        for r in rdmas:
            r.wait()

        c_r = c_rcv[slot].astype(jnp.float32)
        k_b[...] = k_b[...] + _dot(c_r, wuk_rcv[...].astype(jnp.float32))
        v_b[...] = v_b[...] + _dot(c_r, wuv_rcv[...].astype(jnp.float32))

    @pl.when(b == 0)
    def _():
        barrier = pltpu.get_barrier_semaphore()
        pl.semaphore_signal(barrier, inc=1, device_id=peer,
                            device_id_type=pl.DeviceIdType.MESH)
        pl.semaphore_wait(barrier, 1)
        step(0)

    @pl.when(b == 1)
    def _():
        step(1)

    kr = kr_b[...]
    for h in range(H):
        q = q_b[:, h * DH:(h + 1) * DH]
        k = k_b[:, h * DH:(h + 1) * DH]
        v = v_b[:, h * DH:(h + 1) * DH]
        qr = qr_b[:, h * DR:(h + 1) * DR]
        s = (_dot(q, k, ((1,), (1,))) + _dot(qr, kr, ((1,), (1,)))) * SCALE
        p = jnp.exp(s)
        denom = jnp.sum(p, axis=-1, keepdims=True)
        o_ref[0, :, h * DH:(h + 1) * DH] = _dot(p, v, ((1,), (0,))) / denom


def _out_body(o_ref, wo_ref, y_ref):
    y_ref[0, :, :] = _dot(o_ref[0], wo_ref[...])


def kernel(x, Wdkv, Wuk, Wuv, Wq, Wqr, Wkr, Wo):
    f32 = jnp.float32
    bf16 = jnp.bfloat16
    vmem = pl.BlockSpec(memory_space=pltpu.VMEM)

    o = pl.pallas_call(
        _mla_body,
        grid=(B,),
        out_shape=jax.ShapeDtypeStruct((B, S, D), f32),
        in_specs=[
            pl.BlockSpec((1, S, D), lambda b: (b, 0, 0)),
            pl.BlockSpec((D, DC_HALF), lambda b: (0, 0)),
            pl.BlockSpec((DC_HALF, D), lambda b: (0, 0)),
            pl.BlockSpec((DC_HALF, D), lambda b: (0, 0)),
            pl.BlockSpec((D, D), lambda b: (0, 0)),
            pl.BlockSpec((D, H * DR), lambda b: (0, 0)),
            pl.BlockSpec((D, DR), lambda b: (0, 0)),
        ],
        out_specs=pl.BlockSpec((1, S, D), lambda b: (b, 0, 0)),
        scratch_shapes=[
            pltpu.VMEM((S, D), f32),
            pltpu.VMEM((S, D), f32),
            pltpu.VMEM((S, D), f32),
            pltpu.VMEM((S, H * DR), f32),
            pltpu.VMEM((S, DR), f32),
            pltpu.VMEM((B, S, DC_HALF), f32),
            pltpu.VMEM((B, S, DC_HALF), bf16),
            pltpu.VMEM((B, S, DC_HALF), bf16),
            pltpu.VMEM((DC_HALF, D), bf16),
            pltpu.VMEM((DC_HALF, D), bf16),
            pltpu.VMEM((DC_HALF, D), bf16),
            pltpu.VMEM((DC_HALF, D), bf16),
            pltpu.SemaphoreType.DMA((4,)),
            pltpu.SemaphoreType.DMA((4,)),
        ],
        compiler_params=pltpu.CompilerParams(
            collective_id=0, vmem_limit_bytes=100 * 1024 * 1024,
            dimension_semantics=("arbitrary",),
        ),
    )(x, Wdkv, Wuk, Wuv, Wq, Wqr, Wkr)

    return pl.pallas_call(
        _out_body,
        grid=(B,),
        out_shape=jax.ShapeDtypeStruct((B, S, D), f32),
        in_specs=[
            pl.BlockSpec((1, S, D), lambda b: (b, 0, 0)),
            pl.BlockSpec((D, D), lambda b: (0, 0)),
        ],
        out_specs=pl.BlockSpec((1, S, D), lambda b: (b, 0, 0)),
        compiler_params=pltpu.CompilerParams(
            vmem_limit_bytes=100 * 1024 * 1024
        ),
    )(o, Wo)
